# baseline (speedup 1.0000x reference)
import functools

import jax
import jax.numpy as jnp
from jax import lax
from jax.experimental import pallas as pl
from jax.experimental.pallas import tpu as pltpu
from jax.experimental.pallas import tpu_sc as plsc

_SEQ_LENS = (2048, 1792, 1536, 1280, 1024, 768, 512, 256)
_D = 1024
_MAXL = 2048
_NC = 2
_ZROWS = 32


def _pad_body(x0, x1, x2, x3, x4, x5, x6, x7, zsrc, out, zbuf, csem, zsem):
    xs = (x0, x1, x2, x3, x4, x5, x6, x7)
    w = lax.axis_index("s") * _NC + lax.axis_index("c")
    for seq in range(8):
        L = _SEQ_LENS[seq]
        for q in range(4):
            c = min(max(L - 512 * q, 0), 512)
            nz = (512 - c) // _ZROWS
            base = seq * _MAXL + q * 512

            @pl.when(w == seq * 4 + q)
            def _(seq=seq, q=q, c=c, nz=nz, base=base):
                if c > 0:
                    pltpu.async_copy(xs[seq].at[pl.ds(q * 512, c), :],
                                     out.at[pl.ds(base, c), :], csem)
                if nz > 0:
                    pltpu.sync_copy(zsrc, zbuf)
                    for k in range(nz):
                        pltpu.async_copy(
                            zbuf,
                            out.at[pl.ds(base + c + k * _ZROWS, _ZROWS), :],
                            zsem)
                if c > 0:
                    pltpu.make_async_copy(xs[seq].at[pl.ds(q * 512, c), :],
                                          out.at[pl.ds(base, c), :],
                                          csem).wait()
                for k in range(nz):
                    pltpu.make_async_copy(
                        zbuf, out.at[pl.ds(base + c, _ZROWS), :], zsem).wait()


@functools.partial(
    pl.kernel,
    out_type=jax.ShapeDtypeStruct((8 * _MAXL, _D), jnp.float32),
    mesh=plsc.VectorSubcoreMesh(core_axis_name="c", subcore_axis_name="s"),
    scratch_types=[
        pltpu.VMEM((_ZROWS, _D), jnp.float32),
        pltpu.SemaphoreType.DMA,
        pltpu.SemaphoreType.DMA,
    ],
)
def _pad_sc(*refs):
    _pad_body(*refs)


def kernel(x0, x1, x2, x3, x4, x5, x6, x7):
    zsrc = jnp.zeros((_ZROWS, _D), jnp.float32)
    out = _pad_sc(x0, x1, x2, x3, x4, x5, x6, x7, zsrc)
    return out.reshape(8, _MAXL, _D)


# speedup vs baseline: 19.3703x; 19.3703x over previous
import functools

import jax
import jax.numpy as jnp
from jax import lax
from jax.experimental import pallas as pl
from jax.experimental.pallas import tpu as pltpu
from jax.experimental.pallas import tpu_sc as plsc

_SEQ_LENS = (2048, 1792, 1536, 1280, 1024, 768, 512, 256)
_D = 1024
_MAXL = 2048
_NC = 2
_NW = 32
_ZROWS = 32

_ZCHUNKS = tuple(i * _MAXL + r for i, L in enumerate(_SEQ_LENS)
                 for r in range(L, _MAXL, _ZROWS))
_ZPW = len(_ZCHUNKS) // _NW

_TCH = 256
_TNB = 6
_CCHUNKS = tuple((i, k * _TCH) for i, L in enumerate(_SEQ_LENS)
                 for k in range(L // _TCH))


def _zero_body(zsrc, out, zbuf, zsem):
    w = lax.axis_index("s") * _NC + lax.axis_index("c")
    pltpu.sync_copy(zsrc, zbuf)
    for j in range(_ZPW):
        for i in range(_NW):
            base = _ZCHUNKS[i * _ZPW + j]

            @pl.when(w == i)
            def _(base=base):
                pltpu.async_copy(zbuf, out.at[pl.ds(base, _ZROWS), :], zsem)
    for j in range(_ZPW):
        for i in range(_NW):
            base = _ZCHUNKS[i * _ZPW + j]

            @pl.when(w == i)
            def _(base=base):
                pltpu.make_async_copy(zbuf, out.at[pl.ds(base, _ZROWS), :],
                                      zsem).wait()


@functools.partial(
    pl.kernel,
    out_type=jax.ShapeDtypeStruct((8 * _MAXL, _D), jnp.float32),
    mesh=plsc.VectorSubcoreMesh(core_axis_name="c", subcore_axis_name="s"),
    scratch_types=[
        pltpu.VMEM((_ZROWS, _D), jnp.float32),
        pltpu.SemaphoreType.DMA,
    ],
)
def _zero_sc(*refs):
    _zero_body(*refs)


def _copy_tc_body(x0, x1, x2, x3, x4, x5, x6, x7, out_in, out, *scratch):
    del out_in
    bufs = scratch[:_TNB]
    rsems, wsems = scratch[_TNB], scratch[_TNB + 1]
    xs = (x0, x1, x2, x3, x4, x5, x6, x7)
    n = len(_CCHUNKS)

    def rd(k, b):
        seq, r0 = _CCHUNKS[k]
        pltpu.make_async_copy(xs[seq].at[pl.ds(r0, _TCH), :], bufs[b],
                              rsems.at[b]).start()

    def wr(k, b):
        seq, r0 = _CCHUNKS[k]
        pltpu.make_async_copy(bufs[b],
                              out.at[pl.ds(seq * _MAXL + r0, _TCH), :],
                              wsems.at[b]).start()

    for k in range(_TNB - 1):
        rd(k, k)
    for k in range(n):
        b = k % _TNB
        seq, r0 = _CCHUNKS[k]
        pltpu.make_async_copy(xs[seq].at[pl.ds(r0, _TCH), :], bufs[b],
                              rsems.at[b]).wait()
        wr(k, b)
        nxt = k + _TNB - 1
        if nxt < n:
            bn = nxt % _TNB
            if k > 0:
                ps, pr = _CCHUNKS[k - 1]
                pltpu.make_async_copy(
                    bufs[bn], out.at[pl.ds(ps * _MAXL + pr, _TCH), :],
                    wsems.at[bn]).wait()
            rd(nxt, bn)
    for k in range(max(0, n - _TNB), n):
        b = k % _TNB
        seq, r0 = _CCHUNKS[k]
        pltpu.make_async_copy(bufs[b],
                              out.at[pl.ds(seq * _MAXL + r0, _TCH), :],
                              wsems.at[b]).wait()


_copy_tc = pl.pallas_call(
    _copy_tc_body,
    out_shape=jax.ShapeDtypeStruct((8 * _MAXL, _D), jnp.float32),
    in_specs=[pl.BlockSpec(memory_space=pl.ANY)] * 9,
    out_specs=pl.BlockSpec(memory_space=pl.ANY),
    input_output_aliases={8: 0},
    scratch_shapes=[pltpu.VMEM((_TCH, _D), jnp.float32)] * _TNB
    + [pltpu.SemaphoreType.DMA((_TNB,)),
       pltpu.SemaphoreType.DMA((_TNB,))],
)


def kernel(x0, x1, x2, x3, x4, x5, x6, x7):
    zsrc = jnp.zeros((_ZROWS, _D), jnp.float32)
    zeroed = _zero_sc(zsrc)
    out = _copy_tc(x0, x1, x2, x3, x4, x5, x6, x7, zeroed)
    return out.reshape(8, _MAXL, _D)
